# Initial kernel scaffold; baseline (speedup 1.0000x reference)
import jax
import jax.numpy as jnp
from jax.experimental import pallas as pl


def kernel(inputs, z, E, W1, Wz, b1, W2, b2):
    raise NotImplementedError("write your pallas kernel here")


# speedup vs baseline: 60.9772x; 60.9772x over previous
import numpy as np
import jax
import jax.numpy as jnp
from jax.experimental import pallas as pl

_VOCAB_TOKENS = ['<pad>', '<start>', '<end>', 'C', 'O', 'N', '(', ')', '[', ']',
                 '=', '#', '%', '1', '2', '3', '4', '5', '6', '7', '8', '9']
_CONSTRAINT_STRENGTH = 0.5


def _token_tables():
    V = len(_VOCAB_TOKENS)
    base = {'(', '[', ')', ']', 'C', 'O', 'N', '=', '#'}
    digit_allowed = base | {'%'}
    nondigit_allowed = base | {str(i) for i in range(1, 10)}
    is_digit = np.zeros(V, np.float32)
    dis_digit = np.ones(V, np.float32)
    dis_nondigit = np.ones(V, np.float32)
    for idx, tok in enumerate(_VOCAB_TOKENS):
        if tok.isdigit():
            is_digit[idx] = 1.0
        if tok in digit_allowed:
            dis_digit[idx] = 0.0
        if tok in nondigit_allowed:
            dis_nondigit[idx] = 0.0
    return is_digit, dis_digit, dis_nondigit


_IS_DIGIT, _DIS_DIGIT, _DIS_NONDIGIT = _token_tables()
_DIGIT_IDS = [i for i in range(len(_VOCAB_TOKENS)) if _IS_DIGIT[i]]


def _block_body(x_ref, z_ref, E_ref, W1_ref, Wz_ref, b1_ref, W2_ref, b2_ref,
                dd_ref, dn_ref, o_ref):
    x = x_ref[...]
    BB, L = x.shape
    D = W1_ref.shape[0]
    last = x[:, L - 1:L]

    h_sum = jnp.zeros((BB, D), jnp.float32)
    n_digit = jnp.zeros((BB, 1), jnp.float32)
    for v in range(len(_VOCAB_TOKENS)):
        cnt = jnp.sum((x == v).astype(jnp.float32), axis=1, keepdims=True)
        h_sum = h_sum + cnt * E_ref[v, :][None, :]
        if v in _DIGIT_IDS:
            n_digit = n_digit + cnt - (last == v).astype(jnp.float32)

    h = h_sum * (1.0 / L)
    pre = (jnp.dot(h, W1_ref[...], preferred_element_type=jnp.float32)
           + jnp.dot(z_ref[...], Wz_ref[...], preferred_element_type=jnp.float32)
           + b1_ref[...])
    h2 = jnp.tanh(pre)
    logits = jnp.dot(h2, W2_ref[...], preferred_element_type=jnp.float32) + b2_ref[...]

    dd = dd_ref[...]
    dn = dn_ref[...]
    mask = n_digit * dd + (jnp.float32(L - 1) - n_digit) * dn
    o_ref[...] = logits - _CONSTRAINT_STRENGTH * mask


def kernel(inputs, z, E, W1, Wz, b1, W2, b2):
    B, L = inputs.shape
    D = W1.shape[0]
    Z = Wz.shape[0]
    V = E.shape[0]
    BB = 512
    grid = (B // BB,)

    dd = jnp.asarray(_DIS_DIGIT).reshape(1, V)
    dn = jnp.asarray(_DIS_NONDIGIT).reshape(1, V)
    b1r = b1.reshape(1, D)
    b2r = b2.reshape(1, V)

    rep = lambda i: (0, 0)
    blk = lambda i: (i, 0)
    return pl.pallas_call(
        _block_body,
        grid=grid,
        in_specs=[
            pl.BlockSpec((BB, L), blk),
            pl.BlockSpec((BB, Z), blk),
            pl.BlockSpec((V, D), rep),
            pl.BlockSpec((D, D), rep),
            pl.BlockSpec((Z, D), rep),
            pl.BlockSpec((1, D), rep),
            pl.BlockSpec((D, V), rep),
            pl.BlockSpec((1, V), rep),
            pl.BlockSpec((1, V), rep),
            pl.BlockSpec((1, V), rep),
        ],
        out_specs=pl.BlockSpec((BB, V), blk),
        out_shape=jax.ShapeDtypeStruct((B, V), jnp.float32),
    )(inputs, z, E, W1, Wz, b1r, W2, b2r, dd, dn)
